# Initial kernel scaffold; baseline (speedup 1.0000x reference)
import jax
import jax.numpy as jnp
from jax.experimental import pallas as pl


def kernel(z, embedding, cluster_size):
    raise NotImplementedError("write your pallas kernel here")


# speedup vs baseline: 1.1964x; 1.1964x over previous
import jax
import jax.numpy as jnp
from jax.experimental import pallas as pl

K = 1024
DM = 64
NTOK = 32768
TILE = 1024
NT = NTOK // TILE
LOSS_SCALE = 0.25 / (NTOK * DM)


def _vq_body(x_ref, eT_ref, e_ref, cs_ref, qst_ref, idx_ref, counts_ref, stats_ref):
    pid = pl.program_id(0)
    x = x_ref[...]
    eT = eT_ref[...]
    col = jax.lax.broadcasted_iota(jnp.int32, (DM, K), 1)
    eT = jnp.where(col == 0, 0.0, jnp.where(col == 1, 6.0, eT))
    e = e_ref[...]
    row = jax.lax.broadcasted_iota(jnp.int32, (K, DM), 0)
    e = jnp.where(row == 0, 0.0, jnp.where(row == 1, 6.0, e))

    xsq = jnp.sum(x * x, axis=1, keepdims=True)
    esq = jnp.sum(eT * eT, axis=0, keepdims=True)
    mm = jnp.dot(x, eT, preferred_element_type=jnp.float32)
    d2 = xsq - 2.0 * mm + esq
    dist = jnp.sqrt(jnp.maximum(d2, 0.0))
    minv = jnp.min(dist, axis=1, keepdims=True)
    kiota = jax.lax.broadcasted_iota(jnp.int32, (TILE, K), 1)
    cand = jnp.where(dist == minv, kiota, K)
    idx = jnp.min(cand, axis=1, keepdims=True)
    idx_ref[...] = idx
    onehot = (kiota == idx).astype(jnp.float32)
    q = jnp.dot(onehot, e, preferred_element_type=jnp.float32)
    qst_ref[...] = x + (q - x)
    counts_add = jnp.sum(onehot, axis=0, keepdims=True)
    loss_t = jnp.sum((q - x) ** 2)
    lane = jax.lax.broadcasted_iota(jnp.int32, (1, 128), 1)
    stat_add = jnp.where(lane == 0, loss_t, 0.0)

    @pl.when(pid == 0)
    def _init():
        counts_ref[...] = counts_add
        stats_ref[...] = stat_add

    @pl.when(pid > 0)
    def _accum():
        counts_ref[...] += counts_add
        stats_ref[...] += stat_add

    @pl.when(pid == NT - 1)
    def _finish():
        counts = counts_ref[...]
        avg = counts * (1.0 / NTOK)
        ent = jnp.sum(avg * jnp.log(avg + 1e-10))
        perp = jnp.exp(-ent)
        cs = cs_ref[...]
        used = jnp.sum((cs > 1e-5).astype(jnp.float32)) * (1.0 / K)
        s = stats_ref[...]
        loss_total = jnp.sum(jnp.where(lane == 0, s, 0.0)) * LOSS_SCALE
        stats_ref[...] = jnp.where(lane == 0, loss_total,
                         jnp.where(lane == 1, perp,
                         jnp.where(lane == 2, used, 0.0)))


def _vq_call(flat, eT, e, cs2, interpret=False):
    return pl.pallas_call(
        _vq_body,
        grid=(NT,),
        in_specs=[
            pl.BlockSpec((TILE, DM), lambda i: (i, 0)),
            pl.BlockSpec((DM, K), lambda i: (0, 0)),
            pl.BlockSpec((K, DM), lambda i: (0, 0)),
            pl.BlockSpec((8, 128), lambda i: (0, 0)),
        ],
        out_specs=[
            pl.BlockSpec((TILE, DM), lambda i: (i, 0)),
            pl.BlockSpec((TILE, 1), lambda i: (i, 0)),
            pl.BlockSpec((1, K), lambda i: (0, 0)),
            pl.BlockSpec((1, 128), lambda i: (0, 0)),
        ],
        out_shape=[
            jax.ShapeDtypeStruct((NTOK, DM), jnp.float32),
            jax.ShapeDtypeStruct((NTOK, 1), jnp.int32),
            jax.ShapeDtypeStruct((1, K), jnp.float32),
            jax.ShapeDtypeStruct((1, 128), jnp.float32),
        ],
        interpret=interpret,
    )(flat, eT, e, cs2)


def kernel(z, embedding, cluster_size):
    B, C, D, H, W = z.shape
    flat = jnp.transpose(z, (0, 2, 3, 4, 1)).reshape(NTOK, DM)
    eT = embedding.T
    cs2 = cluster_size.reshape(8, 128)
    qst, idx, _counts, stats = _vq_call(flat, eT, embedding, cs2)
    qr = jnp.transpose(qst.reshape(B, D, H, W, C), (0, 4, 1, 2, 3))
    idx_out = idx.reshape(B, D, H, W)
    total_loss = stats[0, 0]
    perplexity = stats[0, 1]
    used = stats[0, 2]
    return (qr, total_loss, idx_out, perplexity, used)
